# baseline (speedup 1.0000x reference)
import functools

import jax
import jax.numpy as jnp
from jax import lax
from jax.experimental import pallas as pl
from jax.experimental.pallas import tpu as pltpu
from jax.experimental.pallas import tpu_sc as plsc

NC = 2
NS = 16
NW = NC * NS
NBUF = 4
GS = (104, 96)


@functools.lru_cache(maxsize=None)
def _build_lookup(S, T, V, D):
    R = S // NW
    mesh = plsc.VectorSubcoreMesh(
        core_axis_name="c", subcore_axis_name="s",
        num_cores=NC, num_subcores=NS)

    @functools.partial(
        pl.kernel,
        out_type=jax.ShapeDtypeStruct((S, T, D), jnp.float32),
        mesh=mesh,
        scratch_types=[
            pltpu.VMEM((R, T), jnp.int32),
            [pltpu.VMEM((T, D), jnp.float32)] * NBUF,
            [pltpu.SemaphoreType.DMA] * NBUF,
            [pltpu.SemaphoreType.DMA] * NBUF,
        ],
        compiler_params=pltpu.CompilerParams(use_tc_tiling_on_sc=False),
    )
    def lookup(idx_hbm, table_hbm, out_hbm, idx_v, bufs, gsems, wsems):
        wid = lax.axis_index("s") * NC + lax.axis_index("c")
        base = wid * R
        pltpu.sync_copy(idx_hbm.at[pl.ds(base, R)], idx_v)

        @pl.loop(0, R, step=NBUF)
        def _block(t):
            gh = []
            for b in range(NBUF):
                off = 0
                for g in GS:
                    gh.append(pltpu.async_copy(
                        table_hbm.at[idx_v.at[t + b, pl.ds(off, g)]],
                        bufs[b].at[pl.ds(off, g)], gsems[b]))
                    off += g
            wh = []
            for b in range(NBUF):
                for j in range(len(GS)):
                    gh[b * len(GS) + j].wait()
                wh.append(pltpu.async_copy(
                    bufs[b], out_hbm.at[base + t + b], wsems[b]))
            for h in wh:
                h.wait()

    return lookup


def kernel(concept, table):
    S, T = concept.shape
    V, D = table.shape
    return _build_lookup(S, T, V, D)(concept.astype(jnp.int32), table)


# speedup vs baseline: 1.2563x; 1.2563x over previous
import functools

import jax
import jax.numpy as jnp
from jax import lax
from jax.experimental import pallas as pl
from jax.experimental.pallas import tpu as pltpu
from jax.experimental.pallas import tpu_sc as plsc

NC = 2
NS = 16
NW = NC * NS
NBUF = 4
GS = (104, 96)


@functools.lru_cache(maxsize=None)
def _build_lookup(S, T, V, D):
    R = S // NW
    mesh = plsc.VectorSubcoreMesh(
        core_axis_name="c", subcore_axis_name="s",
        num_cores=NC, num_subcores=NS)

    @functools.partial(
        pl.kernel,
        out_type=jax.ShapeDtypeStruct((S, T, D), jnp.float32),
        mesh=mesh,
        scratch_types=[
            pltpu.VMEM((R, T), jnp.int32),
            [pltpu.VMEM((T, D), jnp.float32)] * NBUF,
            [pltpu.SemaphoreType.DMA] * NBUF,
            [pltpu.SemaphoreType.DMA] * NBUF,
        ],
        compiler_params=pltpu.CompilerParams(use_tc_tiling_on_sc=False),
    )
    def lookup(idx_hbm, table_hbm, out_hbm, idx_v, bufs, gsems, wsems):
        wid = lax.axis_index("s") * NC + lax.axis_index("c")
        base = wid * R
        pltpu.sync_copy(idx_hbm.at[pl.ds(base, R)], idx_v)

        @pl.loop(0, R, step=NBUF)
        def _block(t):
            gh = []
            for b in range(NBUF):
                off = 0
                for g in GS:
                    gh.append(pltpu.async_copy(
                        table_hbm.at[idx_v.at[t + b, pl.ds(off, g)]],
                        bufs[b].at[pl.ds(off, g)], gsems[b]))
                    off += g
            wh = []
            for b in range(NBUF):
                for j in range(len(GS)):
                    gh[b * len(GS) + j].wait()
                wh.append(pltpu.async_copy(
                    bufs[b], out_hbm.at[base + t + b], wsems[b]))
            for h in wh:
                h.wait()

    return lookup


@functools.lru_cache(maxsize=None)
def _build_transpose(S, T, D):
    NST = S // LN
    J = T * D
    NJ = 2
    JB = J // NJ
    TB = T // NJ

    @functools.partial(
        pl.pallas_call,
        grid=(NST, NJ),
        in_specs=[pl.BlockSpec((1, LN, JB), lambda i, j: (i, 0, j))],
        out_specs=pl.BlockSpec((TB, D // 8, 1, 8, LN),
                               lambda i, j: (j, 0, i, 0, 0)),
        out_shape=jax.ShapeDtypeStruct((T, D // 8, NST, 8, LN), jnp.float32),
    )
    def tx(x_ref, o_ref):
        x = x_ref[0]
        o_ref[:, :, 0, :, :] = x.T.reshape(TB, D // 8, 8, LN)

    return tx


LN = 128


def kernel(concept, table):
    S, T = concept.shape
    V, D = table.shape
    rows = _build_lookup(S, T, V, D)(concept.astype(jnp.int32), table)
    x3 = rows.reshape(S // LN, LN, T * D)
    out5 = _build_transpose(S, T, D)(x3)
    return (out5.transpose(2, 4, 0, 1, 3).reshape(S, T, D))


# speedup vs baseline: 1.2757x; 1.0154x over previous
import functools

import jax
import jax.numpy as jnp
from jax import lax
from jax.experimental import pallas as pl
from jax.experimental.pallas import tpu as pltpu
from jax.experimental.pallas import tpu_sc as plsc

NC = 2
NS = 16
NW = NC * NS
NBUF = 4
GS = (104, 96)


@functools.lru_cache(maxsize=None)
def _build_lookup(S, T, V, D):
    R = S // NW
    mesh = plsc.VectorSubcoreMesh(
        core_axis_name="c", subcore_axis_name="s",
        num_cores=NC, num_subcores=NS)

    @functools.partial(
        pl.kernel,
        out_type=jax.ShapeDtypeStruct((S, T, D), jnp.float32),
        mesh=mesh,
        scratch_types=[
            pltpu.VMEM((R, T), jnp.int32),
            [pltpu.VMEM((T, D), jnp.float32)] * NBUF,
            [pltpu.SemaphoreType.DMA] * NBUF,
            [pltpu.SemaphoreType.DMA] * NBUF,
        ],
        compiler_params=pltpu.CompilerParams(use_tc_tiling_on_sc=False),
    )
    def lookup(idx_hbm, table_hbm, out_hbm, idx_v, bufs, gsems, wsems):
        wid = lax.axis_index("s") * NC + lax.axis_index("c")
        base = wid * R
        pltpu.sync_copy(idx_hbm.at[pl.ds(base, R)], idx_v)

        @pl.loop(0, R, step=NBUF)
        def _block(t):
            gh = []
            for b in range(NBUF):
                off = 0
                for g in GS:
                    gh.append(pltpu.async_copy(
                        table_hbm.at[idx_v.at[t + b, pl.ds(off, g)]],
                        bufs[b].at[pl.ds(off, g)], gsems[b]))
                    off += g
            wh = []
            for b in range(NBUF):
                for j in range(len(GS)):
                    gh[b * len(GS) + j].wait()
                wh.append(pltpu.async_copy(
                    bufs[b], out_hbm.at[base + t + b], wsems[b]))
            for h in wh:
                h.wait()

    return lookup


@functools.lru_cache(maxsize=None)
def _build_transpose(S, T, D):
    NST = S // LN
    J = T * D
    NJ = 1
    JB = J // NJ
    TB = T // NJ

    @functools.partial(
        pl.pallas_call,
        grid=(NST, NJ),
        in_specs=[pl.BlockSpec((1, LN, JB), lambda i, j: (i, 0, j))],
        out_specs=pl.BlockSpec((TB, D // 8, 1, 8, LN),
                               lambda i, j: (j, 0, i, 0, 0)),
        out_shape=jax.ShapeDtypeStruct((T, D // 8, NST, 8, LN), jnp.float32),
    )
    def tx(x_ref, o_ref):
        x = x_ref[0]
        o_ref[:, :, 0, :, :] = x.T.reshape(TB, D // 8, 8, LN)

    return tx


LN = 128


def kernel(concept, table):
    S, T = concept.shape
    V, D = table.shape
    rows = _build_lookup(S, T, V, D)(concept.astype(jnp.int32), table)
    x3 = rows.reshape(S // LN, LN, T * D)
    out5 = _build_transpose(S, T, D)(x3)
    return (out5.transpose(2, 4, 0, 1, 3).reshape(S, T, D))
